# baseline (speedup 1.0000x reference)
import functools

import jax
import jax.numpy as jnp
from jax import lax
from jax.experimental import pallas as pl
from jax.experimental.pallas import tpu as pltpu
from jax.experimental.pallas import tpu_sc as plsc

N = 100000
NP = 102400
F32 = jnp.float32
I32 = jnp.int32

_MESH = plsc.VectorSubcoreMesh(core_axis_name="c", subcore_axis_name="s")
CB = 16


def _worker_chunks(cid, sid, nchunks_total):
    w = cid * 16 + sid
    q = nchunks_total // 32
    rem = nchunks_total % 32
    base = w * q + jnp.minimum(w, rem)
    n = q + (w < rem).astype(I32)
    return w, base, n


def _sc_deg(rows, edge3, zeros, degp, dst_blk, ones_v, deg_sh, sem_s):
    cid = lax.axis_index("c")
    sid = lax.axis_index("s")
    for k in range(8):
        ones_v[pl.ds(k * 16, 16)] = jnp.ones((16,), F32)

    @pl.when(sid == 0)
    def _():
        pltpu.sync_copy(zeros, deg_sh)

    plsc.subcore_barrier()
    nct = rows // CB
    tail = rows % CB
    w, base, n = _worker_chunks(cid, sid, nct)

    def chunk(c, carry):
        r0 = (base + c) * CB
        pltpu.sync_copy(edge3.at[1, pl.ds(r0, CB)], dst_blk)

        def fire(r, cc):
            pltpu.async_copy(ones_v, deg_sh.at[dst_blk.at[r]], sem_s,
                             add=True)
            return cc

        lax.fori_loop(0, CB, fire, 0)

        def drain(r, cc):
            pltpu.make_async_copy(ones_v, deg_sh.at[dst_blk.at[r]],
                                  sem_s).wait()
            return cc

        lax.fori_loop(0, CB, drain, 0)
        return carry

    lax.fori_loop(0, n, chunk, 0)

    if tail:
        @pl.when(w == 31)
        def _():
            pltpu.sync_copy(edge3.at[1, pl.ds(nct * CB, tail)],
                            dst_blk.at[pl.ds(0, tail)])

            def trow(r, cc):
                pltpu.sync_copy(ones_v, deg_sh.at[dst_blk.at[r]], add=True)
                return cc

            lax.fori_loop(0, tail, trow, 0)

    plsc.subcore_barrier()

    @pl.when(sid == 0)
    def _():
        pltpu.sync_copy(deg_sh, degp.at[cid])


def _tc_rsqrt(degp_ref, xil_ref, s0_ref, s1_ref, dn_ref, y0_ref, y1_ref):
    d = lax.rsqrt(jnp.maximum(degp_ref[0] + degp_ref[1], 1.0))
    dn_ref[...] = d
    xil = xil_ref[...]
    y0_ref[...] = d * jnp.dot(xil, s0_ref[...], preferred_element_type=F32)
    y1_ref[...] = d * jnp.dot(xil, s1_ref[...], preferred_element_type=F32)


def _sc_main(rows, edge3, y0t, y1t, ij, zeros,
             u0p, u1p, cip, cjp,
             src_blk, dst_blk, g0, g1, mi_blk, mj_blk, ijv,
             fvmem, fsmem,
             y0_sh, y1_sh, u0_sh, u1_sh, ci_sh, cj_sh, flag_sh,
             sem_g, sem_s):
    cid = lax.axis_index("c")
    sid = lax.axis_index("s")
    pltpu.sync_copy(ij, ijv)
    lanes = lax.iota(I32, 16)

    for which, src, acc in ((0, y0t, y0_sh), (1, y1t, y1_sh),
                            (2, zeros, u0_sh), (3, zeros, u1_sh),
                            (4, zeros, ci_sh), (5, zeros, cj_sh)):
        @pl.when(sid == which)
        def _(src=src, acc=acc):
            pltpu.sync_copy(src, acc)

    plsc.subcore_barrier()
    nct = rows // CB
    tail = rows % CB
    w, base, n = _worker_chunks(cid, sid, nct)
    ivv = ijv[0, :]
    jvv = ijv[1, :]
    one = jnp.ones((16,), F32)
    zero = jnp.zeros((16,), F32)

    def masks_row(b, r):
        acc = zero
        for k in range(8):
            dv = dst_blk[b, r, pl.ds(k * 16, 16)]
            mi = jnp.where(dv == ivv, one, zero)
            mj = jnp.where(dv == jvv, one, zero)
            mi_blk[b, r, pl.ds(k * 16, 16)] = mi
            mj_blk[b, r, pl.ds(k * 16, 16)] = mj
            acc = acc + mi + mj
        for d in (1, 2, 4, 8):
            acc = acc + acc.at[lanes ^ d].get(mode="promise_in_bounds",
                                              unique_indices=True)
        return acc

    def load_and_fire(c, b):
        r0 = (base + c) * CB
        pltpu.sync_copy(edge3.at[0, pl.ds(r0, CB)], src_blk.at[b])
        pltpu.sync_copy(edge3.at[1, pl.ds(r0, CB)], dst_blk.at[b])

        def fg(r, cc):
            pltpu.async_copy(y0_sh.at[src_blk.at[b, r]], g0.at[b, r], sem_g)
            pltpu.async_copy(y1_sh.at[src_blk.at[b, r]], g1.at[b, r], sem_g)
            return cc

        lax.fori_loop(0, CB, fg, 0)

    def process(c, b, nb):
        def mrow(r, flagvec):
            acc = masks_row(b, r)
            return jnp.where(lanes == r, acc, flagvec)

        flagvec = lax.fori_loop(0, CB, mrow, zero)
        fvmem[...] = flagvec
        pltpu.sync_copy(fvmem, flag_sh.at[pl.ds(sid * 16, 16)])
        pltpu.sync_copy(flag_sh.at[pl.ds(sid * 16, 16)], fsmem)

        def drain_g(r, cc):
            pltpu.make_async_copy(y0_sh.at[src_blk.at[b, r]], g0.at[b, r],
                                  sem_g).wait()
            pltpu.make_async_copy(y1_sh.at[src_blk.at[b, r]], g1.at[b, r],
                                  sem_g).wait()
            return cc

        lax.fori_loop(0, CB, drain_g, 0)

        def fire_s(r, cc):
            pltpu.async_copy(g0.at[b, r], u0_sh.at[dst_blk.at[b, r]], sem_s,
                             add=True)
            pltpu.async_copy(g1.at[b, r], u1_sh.at[dst_blk.at[b, r]], sem_s,
                             add=True)
            return cc

        lax.fori_loop(0, CB, fire_s, 0)

        @pl.when(c + 1 < n)
        def _():
            load_and_fire(c + 1, nb)

        def fire_c(r, cc):
            @pl.when(fsmem[r] != 0.0)
            def _():
                pltpu.sync_copy(mi_blk.at[b, r], ci_sh.at[src_blk.at[b, r]],
                                add=True)
                pltpu.sync_copy(mj_blk.at[b, r], cj_sh.at[src_blk.at[b, r]],
                                add=True)

            return cc

        lax.fori_loop(0, CB, fire_c, 0)

        def drain_s(r, cc):
            pltpu.make_async_copy(g0.at[b, r], u0_sh.at[dst_blk.at[b, r]],
                                  sem_s).wait()
            pltpu.make_async_copy(g1.at[b, r], u1_sh.at[dst_blk.at[b, r]],
                                  sem_s).wait()
            return cc

        lax.fori_loop(0, CB, drain_s, 0)

    @pl.when(n > 0)
    def _():
        load_and_fire(0, 0)

    def pair(cc, carry):
        c0 = 2 * cc

        @pl.when(c0 < n)
        def _():
            process(c0, 0, 1)

        @pl.when(c0 + 1 < n)
        def _():
            process(c0 + 1, 1, 0)

        return carry

    lax.fori_loop(0, (n + 1) // 2, pair, 0)

    if tail:
        @pl.when(w == 31)
        def _():
            pltpu.sync_copy(edge3.at[0, pl.ds(nct * CB, tail)],
                            src_blk.at[0, pl.ds(0, tail)])
            pltpu.sync_copy(edge3.at[1, pl.ds(nct * CB, tail)],
                            dst_blk.at[0, pl.ds(0, tail)])

            def trow(r, cc):
                pltpu.sync_copy(y0_sh.at[src_blk.at[0, r]], g0.at[0, r])
                pltpu.sync_copy(y1_sh.at[src_blk.at[0, r]], g1.at[0, r])
                masks_row(0, r)
                pltpu.sync_copy(g0.at[0, r], u0_sh.at[dst_blk.at[0, r]],
                                add=True)
                pltpu.sync_copy(g1.at[0, r], u1_sh.at[dst_blk.at[0, r]],
                                add=True)
                pltpu.sync_copy(mi_blk.at[0, r], ci_sh.at[src_blk.at[0, r]],
                                add=True)
                pltpu.sync_copy(mj_blk.at[0, r], cj_sh.at[src_blk.at[0, r]],
                                add=True)
                return cc

            lax.fori_loop(0, tail, trow, 0)

    plsc.subcore_barrier()
    for which, acc, out in ((0, u0_sh, u0p), (1, u1_sh, u1p),
                            (2, ci_sh, cip), (3, cj_sh, cjp)):
        @pl.when(sid == which)
        def _(acc=acc, out=out):
            pltpu.sync_copy(acc, out.at[cid])


def _tc_finish(dn_ref, u0_ref, u1_ref, ci_ref, cj_ref,
               dnij_ref, W0_ref, b0_ref, W1_ref, b1_ref,
               fc1W_ref, fc1b_ref, fc2W_ref, fc2b_ref, out_ref):
    d = dn_ref[...]
    t0 = d * (u0_ref[0] + u0_ref[1])
    t1 = d * (u1_ref[0] + u1_ref[1])
    wi = d * (ci_ref[0] + ci_ref[1])
    wj = d * (cj_ref[0] + cj_ref[1])
    P = []
    Q = []
    for f in range(16):
        h = jnp.maximum(t0 * W0_ref[0, f] + t1 * W0_ref[1, f] + b0_ref[0, f],
                        0.0)
        P.append(jnp.sum(wi * h))
        Q.append(jnp.sum(wj * h))
    dni = dnij_ref[0, 0]
    dnj = dnij_ref[0, 1]
    embd = []
    for vals, dsc in ((P, dni), (Q, dnj)):
        for g in range(16):
            a = b1_ref[0, g]
            for f in range(16):
                a = a + dsc * vals[f] * W1_ref[f, g]
            embd.append(jnp.maximum(a, 0.0))
    res = []
    for c in range(2):
        a = fc2b_ref[0, c]
        for hh in range(40):
            r = fc1b_ref[0, hh]
            for k in range(32):
                r = r + embd[k] * fc1W_ref[k, hh]
            a = a + jnp.maximum(r, 0.0) * fc2W_ref[hh, c]
        res.append(a)
    ri = lax.broadcasted_iota(I32, (8, 128), 0)
    li = lax.broadcasted_iota(I32, (8, 128), 1)
    out = jnp.where((ri == 0) & (li == 0), res[0],
                    jnp.where((ri == 0) & (li == 1), res[1], 0.0))
    out_ref[...] = out


def kernel(feature_torch, edge_torch, i, j, W0, b0, W1, b1,
           fc1_W, fc1_b, fc2_W, fc2_b):
    E = edge_torch.shape[1]
    pad_e = (-E) % 128
    if pad_e:
        edge_torch = jnp.pad(edge_torch, ((0, 0), (0, pad_e)),
                             constant_values=N)
    rows = edge_torch.shape[1] // 128
    edge3 = edge_torch.reshape(2, rows, 128)

    zeros = jnp.zeros((NP,), F32)
    xil = jnp.pad(feature_torch.reshape(2 * N), (0, 2 * (NP - N)))
    xil = xil.reshape(800, 256)
    ij = jnp.stack([jnp.full((16,), i, I32), jnp.full((16,), j, I32)])
    lane = jnp.arange(128)
    s0 = jnp.zeros((256, 128), F32).at[2 * lane, lane].set(1.0)
    s1 = jnp.zeros((256, 128), F32).at[2 * lane + 1, lane].set(1.0)

    degp = pl.kernel(
        functools.partial(_sc_deg, rows),
        out_type=jax.ShapeDtypeStruct((2, NP), F32),
        mesh=_MESH,
        scratch_types=[
            pltpu.VMEM((CB, 128), I32),
            pltpu.VMEM((128,), F32),
            pltpu.VMEM_SHARED((NP,), F32),
            pltpu.SemaphoreType.DMA,
        ],
    )(edge3, zeros)

    dn2, y0t, y1t = pl.pallas_call(
        _tc_rsqrt,
        out_shape=[jax.ShapeDtypeStruct((800, 128), F32)] * 3,
    )(degp.reshape(2, 800, 128), xil, s0, s1)
    dnp = dn2.reshape(NP)

    u0p, u1p, cip, cjp = pl.kernel(
        functools.partial(_sc_main, rows),
        out_type=[jax.ShapeDtypeStruct((2, NP), F32)] * 4,
        mesh=_MESH,
        scratch_types=[
            pltpu.VMEM((2, CB, 128), I32),
            pltpu.VMEM((2, CB, 128), I32),
            pltpu.VMEM((2, CB, 128), F32),
            pltpu.VMEM((2, CB, 128), F32),
            pltpu.VMEM((2, CB, 128), F32),
            pltpu.VMEM((2, CB, 128), F32),
            pltpu.VMEM((2, 16), I32),
            pltpu.VMEM((16,), F32),
            pltpu.SMEM((16,), F32),
            pltpu.VMEM_SHARED((NP,), F32),
            pltpu.VMEM_SHARED((NP,), F32),
            pltpu.VMEM_SHARED((NP,), F32),
            pltpu.VMEM_SHARED((NP,), F32),
            pltpu.VMEM_SHARED((NP,), F32),
            pltpu.VMEM_SHARED((NP,), F32),
            pltpu.VMEM_SHARED((256,), F32),
            pltpu.SemaphoreType.DMA,
            pltpu.SemaphoreType.DMA,
        ],
    )(edge3, y0t.reshape(NP), y1t.reshape(NP), ij, zeros)

    dnij = jnp.stack([dnp[i], dnp[j]]).reshape(1, 2)
    smem = pl.BlockSpec(memory_space=pltpu.SMEM)
    vmem = pl.BlockSpec(memory_space=pltpu.VMEM)
    out_pad = pl.pallas_call(
        _tc_finish,
        out_shape=jax.ShapeDtypeStruct((8, 128), F32),
        in_specs=[vmem] * 5 + [smem] * 9,
        out_specs=vmem,
    )(dn2,
      u0p.reshape(2, 800, 128), u1p.reshape(2, 800, 128),
      cip.reshape(2, 800, 128), cjp.reshape(2, 800, 128),
      dnij, W0, b0.reshape(1, 16), W1, b1.reshape(1, 16),
      fc1_W, fc1_b.reshape(1, 40), fc2_W, fc2_b.reshape(1, 2))
    return out_pad[0, :2]


# speedup vs baseline: 1.0568x; 1.0568x over previous
import functools

import jax
import jax.numpy as jnp
from jax import lax
from jax.experimental import pallas as pl
from jax.experimental.pallas import tpu as pltpu
from jax.experimental.pallas import tpu_sc as plsc

N = 100000
NP = 102400
F32 = jnp.float32
I32 = jnp.int32

_MESH = plsc.VectorSubcoreMesh(core_axis_name="c", subcore_axis_name="s")
CB = 16


def _worker_chunks(cid, sid, nchunks_total):
    w = cid * 16 + sid
    q = nchunks_total // 32
    rem = nchunks_total % 32
    base = w * q + jnp.minimum(w, rem)
    n = q + (w < rem).astype(I32)
    return w, base, n


def _sc_deg(rows, edge3, zeros, degp, dst_blk, ones_v, deg_sh, sem_s):
    cid = lax.axis_index("c")
    sid = lax.axis_index("s")
    for k in range(8):
        ones_v[pl.ds(k * 16, 16)] = jnp.ones((16,), F32)

    @pl.when(sid == 0)
    def _():
        pltpu.sync_copy(zeros, deg_sh)

    plsc.subcore_barrier()
    nct = rows // CB
    tail = rows % CB
    w, base, n = _worker_chunks(cid, sid, nct)

    def load(c, b):
        r0 = (base + c) * CB
        pltpu.sync_copy(edge3.at[1, pl.ds(r0, CB)], dst_blk.at[b])

    def process(c, b, nb):
        def fire(r, cc):
            pltpu.async_copy(ones_v, deg_sh.at[dst_blk.at[b, r]], sem_s,
                             add=True)
            return cc

        lax.fori_loop(0, CB, fire, 0)

        @pl.when(c + 1 < n)
        def _():
            load(c + 1, nb)

        def drain(r, cc):
            pltpu.make_async_copy(ones_v, deg_sh.at[dst_blk.at[b, r]],
                                  sem_s).wait()
            return cc

        lax.fori_loop(0, CB, drain, 0)

    @pl.when(n > 0)
    def _():
        load(0, 0)

    def pair(cc, carry):
        c0 = 2 * cc

        @pl.when(c0 < n)
        def _():
            process(c0, 0, 1)

        @pl.when(c0 + 1 < n)
        def _():
            process(c0 + 1, 1, 0)

        return carry

    lax.fori_loop(0, (n + 1) // 2, pair, 0)

    if tail:
        @pl.when(w == 31)
        def _():
            pltpu.sync_copy(edge3.at[1, pl.ds(nct * CB, tail)],
                            dst_blk.at[0, pl.ds(0, tail)])

            def trow(r, cc):
                pltpu.sync_copy(ones_v, deg_sh.at[dst_blk.at[0, r]],
                                add=True)
                return cc

            lax.fori_loop(0, tail, trow, 0)

    plsc.subcore_barrier()

    @pl.when(sid == 0)
    def _():
        pltpu.sync_copy(deg_sh, degp.at[cid])


def _tc_rsqrt(degp_ref, xil_ref, s0_ref, s1_ref, dn_ref, yp_ref):
    d = lax.rsqrt(jnp.maximum(degp_ref[0] + degp_ref[1], 1.0))
    dn_ref[...] = d
    xil = xil_ref[...]
    y0 = d * jnp.dot(xil, s0_ref[...], preferred_element_type=F32)
    y1 = d * jnp.dot(xil, s1_ref[...], preferred_element_type=F32)
    y0i = lax.bitcast_convert_type(y0, I32) + 0x8000
    y1i = lax.bitcast_convert_type(y1, I32) + 0x8000
    yp_ref[...] = (y0i & (-65536)) | lax.shift_right_logical(y1i, 16)


def _sc_main(rows, edge3, ypt, ij, zeros,
             u0p, u1p, cip, cjp,
             src_blk, dst_blk, gp, g0, g1, mi_blk, mj_blk, ijv,
             fvmem, fsmem,
             yp_sh, u0_sh, u1_sh, ci_sh, cj_sh, flag_sh,
             sem_g, sem_s):
    cid = lax.axis_index("c")
    sid = lax.axis_index("s")
    pltpu.sync_copy(ij, ijv)
    lanes = lax.iota(I32, 16)
    himask = jnp.full((16,), -65536, I32)

    for which, src, acc in ((0, ypt, yp_sh), (1, zeros, u0_sh),
                            (2, zeros, u1_sh), (3, zeros, ci_sh),
                            (4, zeros, cj_sh)):
        @pl.when(sid == which)
        def _(src=src, acc=acc):
            pltpu.sync_copy(src, acc)

    plsc.subcore_barrier()
    nct = rows // CB
    tail = rows % CB
    w, base, n = _worker_chunks(cid, sid, nct)
    ivv = ijv[0, :]
    jvv = ijv[1, :]
    one = jnp.ones((16,), F32)
    zero = jnp.zeros((16,), F32)

    def masks_row(b, r):
        acc = zero
        for k in range(8):
            dv = dst_blk[b, r, pl.ds(k * 16, 16)]
            mi = jnp.where(dv == ivv, one, zero)
            mj = jnp.where(dv == jvv, one, zero)
            mi_blk[b, r, pl.ds(k * 16, 16)] = mi
            mj_blk[b, r, pl.ds(k * 16, 16)] = mj
            acc = acc + mi + mj
        for d in (1, 2, 4, 8):
            acc = acc + acc.at[lanes ^ d].get(mode="promise_in_bounds",
                                              unique_indices=True)
        return acc

    def load_and_fire(c, b):
        r0 = (base + c) * CB
        pltpu.sync_copy(edge3.at[0, pl.ds(r0, CB)], src_blk.at[b])
        pltpu.sync_copy(edge3.at[1, pl.ds(r0, CB)], dst_blk.at[b])

        def fg(r, cc):
            pltpu.async_copy(yp_sh.at[src_blk.at[b, r]], gp.at[b, r], sem_g)
            return cc

        lax.fori_loop(0, CB, fg, 0)

    def process(c, b, nb):
        def mrow(r, flagvec):
            acc = masks_row(b, r)
            return jnp.where(lanes == r, acc, flagvec)

        flagvec = lax.fori_loop(0, CB, mrow, zero)
        fvmem[...] = flagvec
        pltpu.sync_copy(fvmem, flag_sh.at[pl.ds(sid * 16, 16)])
        pltpu.sync_copy(flag_sh.at[pl.ds(sid * 16, 16)], fsmem)

        def drain_g(r, cc):
            pltpu.make_async_copy(yp_sh.at[src_blk.at[b, r]], gp.at[b, r],
                                  sem_g).wait()
            return cc

        lax.fori_loop(0, CB, drain_g, 0)

        def unpack_row(r, cc):
            for k in range(8):
                v = gp[b, r, pl.ds(k * 16, 16)]
                g0[b, r, pl.ds(k * 16, 16)] = lax.bitcast_convert_type(
                    v & himask, F32)
                g1[b, r, pl.ds(k * 16, 16)] = lax.bitcast_convert_type(
                    lax.shift_left(v, 16), F32)
            return cc

        lax.fori_loop(0, CB, unpack_row, 0)

        def fire_s(r, cc):
            pltpu.async_copy(g0.at[b, r], u0_sh.at[dst_blk.at[b, r]], sem_s,
                             add=True)
            pltpu.async_copy(g1.at[b, r], u1_sh.at[dst_blk.at[b, r]], sem_s,
                             add=True)
            return cc

        lax.fori_loop(0, CB, fire_s, 0)

        @pl.when(c + 1 < n)
        def _():
            load_and_fire(c + 1, nb)

        def fire_c(r, cc):
            @pl.when(fsmem[r] != 0.0)
            def _():
                pltpu.sync_copy(mi_blk.at[b, r], ci_sh.at[src_blk.at[b, r]],
                                add=True)
                pltpu.sync_copy(mj_blk.at[b, r], cj_sh.at[src_blk.at[b, r]],
                                add=True)

            return cc

        lax.fori_loop(0, CB, fire_c, 0)

        def drain_s(r, cc):
            pltpu.make_async_copy(g0.at[b, r], u0_sh.at[dst_blk.at[b, r]],
                                  sem_s).wait()
            pltpu.make_async_copy(g1.at[b, r], u1_sh.at[dst_blk.at[b, r]],
                                  sem_s).wait()
            return cc

        lax.fori_loop(0, CB, drain_s, 0)

    @pl.when(n > 0)
    def _():
        load_and_fire(0, 0)

    def pair(cc, carry):
        c0 = 2 * cc

        @pl.when(c0 < n)
        def _():
            process(c0, 0, 1)

        @pl.when(c0 + 1 < n)
        def _():
            process(c0 + 1, 1, 0)

        return carry

    lax.fori_loop(0, (n + 1) // 2, pair, 0)

    if tail:
        @pl.when(w == 31)
        def _():
            pltpu.sync_copy(edge3.at[0, pl.ds(nct * CB, tail)],
                            src_blk.at[0, pl.ds(0, tail)])
            pltpu.sync_copy(edge3.at[1, pl.ds(nct * CB, tail)],
                            dst_blk.at[0, pl.ds(0, tail)])

            def trow(r, cc):
                pltpu.sync_copy(yp_sh.at[src_blk.at[0, r]], gp.at[0, r])
                for k in range(8):
                    v = gp[0, r, pl.ds(k * 16, 16)]
                    g0[0, r, pl.ds(k * 16, 16)] = lax.bitcast_convert_type(
                        v & himask, F32)
                    g1[0, r, pl.ds(k * 16, 16)] = lax.bitcast_convert_type(
                        lax.shift_left(v, 16), F32)
                masks_row(0, r)
                pltpu.sync_copy(g0.at[0, r], u0_sh.at[dst_blk.at[0, r]],
                                add=True)
                pltpu.sync_copy(g1.at[0, r], u1_sh.at[dst_blk.at[0, r]],
                                add=True)
                pltpu.sync_copy(mi_blk.at[0, r], ci_sh.at[src_blk.at[0, r]],
                                add=True)
                pltpu.sync_copy(mj_blk.at[0, r], cj_sh.at[src_blk.at[0, r]],
                                add=True)
                return cc

            lax.fori_loop(0, tail, trow, 0)

    plsc.subcore_barrier()
    for which, acc, out in ((0, u0_sh, u0p), (1, u1_sh, u1p),
                            (2, ci_sh, cip), (3, cj_sh, cjp)):
        @pl.when(sid == which)
        def _(acc=acc, out=out):
            pltpu.sync_copy(acc, out.at[cid])


def _tc_finish(dn_ref, u0_ref, u1_ref, ci_ref, cj_ref,
               dnij_ref, W0_ref, b0_ref, W1_ref, b1_ref,
               fc1W_ref, fc1b_ref, fc2W_ref, fc2b_ref, out_ref):
    d = dn_ref[...]
    t0 = d * (u0_ref[0] + u0_ref[1])
    t1 = d * (u1_ref[0] + u1_ref[1])
    wi = d * (ci_ref[0] + ci_ref[1])
    wj = d * (cj_ref[0] + cj_ref[1])
    P = []
    Q = []
    for f in range(16):
        h = jnp.maximum(t0 * W0_ref[0, f] + t1 * W0_ref[1, f] + b0_ref[0, f],
                        0.0)
        P.append(jnp.sum(wi * h))
        Q.append(jnp.sum(wj * h))
    dni = dnij_ref[0, 0]
    dnj = dnij_ref[0, 1]
    embd = []
    for vals, dsc in ((P, dni), (Q, dnj)):
        for g in range(16):
            a = b1_ref[0, g]
            for f in range(16):
                a = a + dsc * vals[f] * W1_ref[f, g]
            embd.append(jnp.maximum(a, 0.0))
    res = []
    for c in range(2):
        a = fc2b_ref[0, c]
        for hh in range(40):
            r = fc1b_ref[0, hh]
            for k in range(32):
                r = r + embd[k] * fc1W_ref[k, hh]
            a = a + jnp.maximum(r, 0.0) * fc2W_ref[hh, c]
        res.append(a)
    ri = lax.broadcasted_iota(I32, (8, 128), 0)
    li = lax.broadcasted_iota(I32, (8, 128), 1)
    out = jnp.where((ri == 0) & (li == 0), res[0],
                    jnp.where((ri == 0) & (li == 1), res[1], 0.0))
    out_ref[...] = out


def kernel(feature_torch, edge_torch, i, j, W0, b0, W1, b1,
           fc1_W, fc1_b, fc2_W, fc2_b):
    E = edge_torch.shape[1]
    pad_e = (-E) % 128
    if pad_e:
        edge_torch = jnp.pad(edge_torch, ((0, 0), (0, pad_e)),
                             constant_values=N)
    rows = edge_torch.shape[1] // 128
    edge3 = edge_torch.reshape(2, rows, 128)

    zeros = jnp.zeros((NP,), F32)
    xil = jnp.pad(feature_torch.reshape(2 * N), (0, 2 * (NP - N)))
    xil = xil.reshape(800, 256)
    ij = jnp.stack([jnp.full((16,), i, I32), jnp.full((16,), j, I32)])
    lane = jnp.arange(128)
    s0 = jnp.zeros((256, 128), F32).at[2 * lane, lane].set(1.0)
    s1 = jnp.zeros((256, 128), F32).at[2 * lane + 1, lane].set(1.0)

    degp = pl.kernel(
        functools.partial(_sc_deg, rows),
        out_type=jax.ShapeDtypeStruct((2, NP), F32),
        mesh=_MESH,
        scratch_types=[
            pltpu.VMEM((2, CB, 128), I32),
            pltpu.VMEM((128,), F32),
            pltpu.VMEM_SHARED((NP,), F32),
            pltpu.SemaphoreType.DMA,
        ],
    )(edge3, zeros)

    dn2, ypt = pl.pallas_call(
        _tc_rsqrt,
        out_shape=[jax.ShapeDtypeStruct((800, 128), F32),
                   jax.ShapeDtypeStruct((800, 128), I32)],
    )(degp.reshape(2, 800, 128), xil, s0, s1)
    dnp = dn2.reshape(NP)

    u0p, u1p, cip, cjp = pl.kernel(
        functools.partial(_sc_main, rows),
        out_type=[jax.ShapeDtypeStruct((2, NP), F32)] * 4,
        mesh=_MESH,
        scratch_types=[
            pltpu.VMEM((2, CB, 128), I32),
            pltpu.VMEM((2, CB, 128), I32),
            pltpu.VMEM((2, CB, 128), I32),
            pltpu.VMEM((2, CB, 128), F32),
            pltpu.VMEM((2, CB, 128), F32),
            pltpu.VMEM((2, CB, 128), F32),
            pltpu.VMEM((2, CB, 128), F32),
            pltpu.VMEM((2, 16), I32),
            pltpu.VMEM((16,), F32),
            pltpu.SMEM((16,), F32),
            pltpu.VMEM_SHARED((NP,), I32),
            pltpu.VMEM_SHARED((NP,), F32),
            pltpu.VMEM_SHARED((NP,), F32),
            pltpu.VMEM_SHARED((NP,), F32),
            pltpu.VMEM_SHARED((NP,), F32),
            pltpu.VMEM_SHARED((256,), F32),
            pltpu.SemaphoreType.DMA,
            pltpu.SemaphoreType.DMA,
        ],
    )(edge3, ypt.reshape(NP), ij, zeros)

    dnij = jnp.stack([dnp[i], dnp[j]]).reshape(1, 2)
    smem = pl.BlockSpec(memory_space=pltpu.SMEM)
    vmem = pl.BlockSpec(memory_space=pltpu.VMEM)
    out_pad = pl.pallas_call(
        _tc_finish,
        out_shape=jax.ShapeDtypeStruct((8, 128), F32),
        in_specs=[vmem] * 5 + [smem] * 9,
        out_specs=vmem,
    )(dn2,
      u0p.reshape(2, 800, 128), u1p.reshape(2, 800, 128),
      cip.reshape(2, 800, 128), cjp.reshape(2, 800, 128),
      dnij, W0, b0.reshape(1, 16), W1, b1.reshape(1, 16),
      fc1_W, fc1_b.reshape(1, 40), fc2_W, fc2_b.reshape(1, 2))
    return out_pad[0, :2]
